# baseline (speedup 1.0000x reference)
import functools

import jax
import jax.numpy as jnp
from jax import lax
from jax.experimental import pallas as pl
from jax.experimental.pallas import tpu as pltpu
from jax.experimental.pallas import tpu_sc as plsc

N_ATOMS = 1048576
EMB = 128
NUM_CORES = 2
NUM_SUBCORES = 16
NUM_WORKERS = NUM_CORES * NUM_SUBCORES
B_PER_W = N_ATOMS // NUM_WORKERS
IDX_MINOR = 128
IDX_ROWS = B_PER_W // IDX_MINOR
BUF_ROWS = 128
NBUF = 4
N_UNITS = B_PER_W // BUF_ROWS
TAB_ROWS = 104

_mesh = plsc.VectorSubcoreMesh(core_axis_name="c", subcore_axis_name="s")


@functools.partial(
    pl.kernel,
    out_type=jax.ShapeDtypeStruct((N_ATOMS, EMB), jnp.float32),
    mesh=_mesh,
    scratch_types=[
        pltpu.VMEM((IDX_ROWS, IDX_MINOR), jnp.int32),
        pltpu.VMEM((NBUF, BUF_ROWS, EMB), jnp.float32),
        pltpu.VMEM_SHARED((TAB_ROWS, EMB), jnp.float32),
        pltpu.SemaphoreType.DMA,
        pltpu.SemaphoreType.DMA,
        pltpu.SemaphoreType.DMA,
        pltpu.SemaphoreType.DMA,
        pltpu.SemaphoreType.DMA,
        pltpu.SemaphoreType.DMA,
        pltpu.SemaphoreType.DMA,
        pltpu.SemaphoreType.DMA,
        pltpu.SemaphoreType.DMA,
        pltpu.SemaphoreType.DMA,
        pltpu.SemaphoreType.DMA,
        pltpu.SemaphoreType.DMA,
        pltpu.SemaphoreType.DMA,
        pltpu.SemaphoreType.DMA,
        pltpu.SemaphoreType.DMA,
        pltpu.SemaphoreType.DMA,
        pltpu.SemaphoreType.DMA,

    ],
)
def _emb_lookup(
    table_hbm, z2d_hbm, out_hbm,
    idx_v, rows_v, table_sh, i_sem,
    g_sem0, g_sem1, g_sem2, g_sem3, g_sem4, g_sem5, g_sem6, g_sem7,
    st_sem0, st_sem1, st_sem2, st_sem3, st_sem4, st_sem5, st_sem6, st_sem7,
):
    wid = lax.axis_index("s") * NUM_CORES + lax.axis_index("c")
    base = wid * B_PER_W
    g_sems = (g_sem0, g_sem1, g_sem2, g_sem3, g_sem4, g_sem5, g_sem6, g_sem7)
    st_sems = (st_sem0, st_sem1, st_sem2, st_sem3, st_sem4, st_sem5, st_sem6, st_sem7)

    idx_row = pl.multiple_of(wid * IDX_ROWS, IDX_ROWS)
    idx_cp = pltpu.async_copy(
        z2d_hbm.at[pl.ds(idx_row, IDX_ROWS)], idx_v, i_sem
    )

    @pl.when(lax.axis_index("s") == 0)
    def _():
        pltpu.sync_copy(table_hbm, table_sh)

    plsc.subcore_barrier()
    idx_cp.wait()

    def issue_gather(u, b):
        pltpu.async_copy(
            table_sh.at[idx_v.at[u]], rows_v.at[b], g_sems[b]
        )

    def wait_gather(b):
        pltpu.make_async_copy(
            table_sh.at[idx_v.at[0]], rows_v.at[b], g_sems[b]
        ).wait()

    def issue_store(u, b):
        pltpu.async_copy(
            rows_v.at[b],
            out_hbm.at[pl.ds(base + u * BUF_ROWS, BUF_ROWS)],
            st_sems[b],
        )

    def wait_store(b):
        pltpu.make_async_copy(
            rows_v.at[b], out_hbm.at[pl.ds(0, BUF_ROWS)], st_sems[b]
        ).wait()

    def body(s, carry):
        for j in range(NBUF):
            u = NBUF * s + j
            b = j
            pb = (j - 2) % NBUF
            if j < 2:
                @pl.when(s > 0)
                def _():
                    wait_store(b)
                    issue_gather(u, b)
                    wait_gather(pb)
                    issue_store(u - 2, pb)
                @pl.when(s == 0)
                def _():
                    issue_gather(u, b)
            else:
                @pl.when(s > 0)
                def _():
                    wait_store(b)
                issue_gather(u, b)
                wait_gather(pb)
                issue_store(u - 2, pb)
        return carry

    lax.fori_loop(0, N_UNITS // NBUF, body, 0)
    for k in (2, 1):
        wait_gather(NBUF - k)
        issue_store(N_UNITS - k, NBUF - k)
    for b in range(NBUF):
        wait_store(b)


def kernel(Z, embeddings):
    table = jnp.concatenate(
        [
            jnp.zeros((1, EMB), embeddings.dtype),
            embeddings,
            jnp.zeros((TAB_ROWS - 1 - embeddings.shape[0], EMB), embeddings.dtype),
        ],
        axis=0,
    )
    z2d = Z.astype(jnp.int32).reshape(N_ATOMS // IDX_MINOR, IDX_MINOR)
    return _emb_lookup(table, z2d)


# speedup vs baseline: 1.0134x; 1.0134x over previous
import functools

import jax
import jax.numpy as jnp
from jax import lax
from jax.experimental import pallas as pl
from jax.experimental.pallas import tpu as pltpu
from jax.experimental.pallas import tpu_sc as plsc

N_ATOMS = 1048576
EMB = 128
NUM_CORES = 2
NUM_SUBCORES = 16
NUM_WORKERS = NUM_CORES * NUM_SUBCORES
B_PER_W = N_ATOMS // NUM_WORKERS
IDX_MINOR = 128
IDX_ROWS = B_PER_W // IDX_MINOR
BUF_ROWS = 64
NBUF = 8
N_UNITS = B_PER_W // BUF_ROWS
TAB_ROWS = 104

_mesh = plsc.VectorSubcoreMesh(core_axis_name="c", subcore_axis_name="s")


@functools.partial(
    pl.kernel,
    out_type=jax.ShapeDtypeStruct((N_ATOMS, EMB), jnp.float32),
    mesh=_mesh,
    scratch_types=[
        pltpu.VMEM((IDX_ROWS, IDX_MINOR), jnp.int32),
        pltpu.VMEM((NBUF, BUF_ROWS, EMB), jnp.float32),
        pltpu.VMEM_SHARED((TAB_ROWS, EMB), jnp.float32),
        pltpu.SemaphoreType.DMA,
        pltpu.SemaphoreType.DMA,
        pltpu.SemaphoreType.DMA,
        pltpu.SemaphoreType.DMA,
        pltpu.SemaphoreType.DMA,
        pltpu.SemaphoreType.DMA,
        pltpu.SemaphoreType.DMA,
        pltpu.SemaphoreType.DMA,
        pltpu.SemaphoreType.DMA,
        pltpu.SemaphoreType.DMA,
        pltpu.SemaphoreType.DMA,
        pltpu.SemaphoreType.DMA,
        pltpu.SemaphoreType.DMA,
        pltpu.SemaphoreType.DMA,
        pltpu.SemaphoreType.DMA,
        pltpu.SemaphoreType.DMA,
        pltpu.SemaphoreType.DMA,

    ],
)
def _emb_lookup(
    table_hbm, z2d_hbm, out_hbm,
    idx_v, rows_v, table_sh, i_sem,
    g_sem0, g_sem1, g_sem2, g_sem3, g_sem4, g_sem5, g_sem6, g_sem7,
    st_sem0, st_sem1, st_sem2, st_sem3, st_sem4, st_sem5, st_sem6, st_sem7,
):
    wid = lax.axis_index("s") * NUM_CORES + lax.axis_index("c")
    base = wid * B_PER_W
    g_sems = (g_sem0, g_sem1, g_sem2, g_sem3, g_sem4, g_sem5, g_sem6, g_sem7)
    st_sems = (st_sem0, st_sem1, st_sem2, st_sem3, st_sem4, st_sem5, st_sem6, st_sem7)

    idx_row = pl.multiple_of(wid * IDX_ROWS, IDX_ROWS)
    idx_cp = pltpu.async_copy(
        z2d_hbm.at[pl.ds(idx_row, IDX_ROWS)], idx_v, i_sem
    )

    @pl.when(lax.axis_index("s") == 0)
    def _():
        pltpu.sync_copy(table_hbm, table_sh)

    plsc.subcore_barrier()
    idx_cp.wait()

    def issue_gather(u, b):
        pltpu.async_copy(
            table_sh.at[idx_v.at[u // 2, pl.ds((u % 2) * BUF_ROWS, BUF_ROWS)]],
            rows_v.at[b],
            g_sems[b],
        )

    def wait_gather(b):
        pltpu.make_async_copy(
            table_sh.at[idx_v.at[0, pl.ds(0, BUF_ROWS)]], rows_v.at[b], g_sems[b]
        ).wait()

    def issue_store(u, b):
        pltpu.async_copy(
            rows_v.at[b],
            out_hbm.at[pl.ds(base + u * BUF_ROWS, BUF_ROWS)],
            st_sems[b],
        )

    def wait_store(b):
        pltpu.make_async_copy(
            rows_v.at[b], out_hbm.at[pl.ds(0, BUF_ROWS)], st_sems[b]
        ).wait()

    def body(s, carry):
        for j in range(NBUF):
            u = NBUF * s + j
            b = j
            pb = (j - 2) % NBUF
            if j < 2:
                @pl.when(s > 0)
                def _():
                    wait_store(b)
                    issue_gather(u, b)
                    wait_gather(pb)
                    issue_store(u - 2, pb)
                @pl.when(s == 0)
                def _():
                    issue_gather(u, b)
            else:
                @pl.when(s > 0)
                def _():
                    wait_store(b)
                issue_gather(u, b)
                wait_gather(pb)
                issue_store(u - 2, pb)
        return carry

    lax.fori_loop(0, N_UNITS // NBUF, body, 0)
    for k in (2, 1):
        wait_gather(NBUF - k)
        issue_store(N_UNITS - k, NBUF - k)
    for b in range(NBUF):
        wait_store(b)


def kernel(Z, embeddings):
    table = jnp.concatenate(
        [
            jnp.zeros((1, EMB), embeddings.dtype),
            embeddings,
            jnp.zeros((TAB_ROWS - 1 - embeddings.shape[0], EMB), embeddings.dtype),
        ],
        axis=0,
    )
    z2d = Z.astype(jnp.int32).reshape(N_ATOMS // IDX_MINOR, IDX_MINOR)
    return _emb_lookup(table, z2d)
